# Initial kernel scaffold; baseline (speedup 1.0000x reference)
import jax
import jax.numpy as jnp
from jax.experimental import pallas as pl


def kernel(x, token_table, pos_table):
    raise NotImplementedError("write your pallas kernel here")


# speedup vs baseline: 4.2431x; 4.2431x over previous
import functools

import jax
import jax.numpy as jnp
from jax import lax
from jax.experimental import pallas as pl
from jax.experimental.pallas import tpu as pltpu
from jax.experimental.pallas import tpu_sc as plsc

EMBED = 128
SEQ = 200
LANES = 16
NC = 2
NS = 16
NW = NC * NS
IDXW = 100


@functools.partial(jax.jit, static_argnums=(3, 4))
def _tok_pos_embed(idx2d, token_table, pos_table, total, maxlen):
    seq_per_w = total // NW // SEQ
    rows_per_w = total // NW

    mesh = plsc.VectorSubcoreMesh(core_axis_name="c", subcore_axis_name="s")

    @functools.partial(
        pl.kernel,
        mesh=mesh,
        out_type=jax.ShapeDtypeStruct((total, EMBED), jnp.float32),
        scratch_types=[
            pltpu.VMEM((2, IDXW), jnp.int32),
            pltpu.VMEM((SEQ, EMBED), jnp.float32),
            pltpu.VMEM((maxlen, EMBED), jnp.float32),
            pltpu.SemaphoreType.DMA,
        ],
    )
    def emb(idx_hbm, tok_hbm, pos_hbm, out_hbm, idx_v, rows_v, pos_v, sem):
        wid = lax.axis_index("s") * NC + lax.axis_index("c")
        pltpu.sync_copy(pos_hbm, pos_v)

        def chunk_body(c, carry):
            irow = wid * (rows_per_w // IDXW) + c * (SEQ // IDXW)
            out_base = wid * rows_per_w + c * SEQ
            pltpu.sync_copy(idx_hbm.at[pl.ds(irow, 2)], idx_v)
            cp0 = pltpu.async_copy(
                tok_hbm.at[idx_v.at[0]], rows_v.at[pl.ds(0, IDXW)], sem)
            cp1 = pltpu.async_copy(
                tok_hbm.at[idx_v.at[1]], rows_v.at[pl.ds(IDXW, IDXW)], sem)
            cp0.wait()
            cp1.wait()

            def row_body(i, rcarry):
                for j in range(EMBED // LANES):
                    sl = pl.ds(j * LANES, LANES)
                    rows_v[i, sl] = rows_v[i, sl] + pos_v[i, sl]
                return rcarry

            lax.fori_loop(0, SEQ, row_body, 0)
            pltpu.sync_copy(rows_v, out_hbm.at[pl.ds(out_base, SEQ)])
            return carry

        lax.fori_loop(0, seq_per_w, chunk_body, 0)

    return emb(idx2d, token_table, pos_table)


def kernel(x, token_table, pos_table):
    batch, seq = x.shape
    total = batch * seq
    idx2d = x.reshape(total // IDXW, IDXW).astype(jnp.int32)
    out = _tok_pos_embed(idx2d, token_table, pos_table, total,
                         pos_table.shape[0])
    return out.reshape(batch, seq, EMBED)


# speedup vs baseline: 8.9306x; 2.1047x over previous
import functools

import jax
import jax.numpy as jnp
from jax import lax
from jax.experimental import pallas as pl
from jax.experimental.pallas import tpu as pltpu
from jax.experimental.pallas import tpu_sc as plsc

EMBED = 128
SEQ = 200
NC = 2
NS = 16
NW = NC * NS
IDXW = 100
NBUF = 2


@functools.partial(jax.jit, static_argnums=(3,))
def _tok_pos_embed(idx2d, token_table, pos_table, total):
    seq_per_w = total // NW // SEQ
    rows_per_w = total // NW
    irows_per_w = rows_per_w // IDXW

    mesh = plsc.VectorSubcoreMesh(core_axis_name="c", subcore_axis_name="s")

    @functools.partial(
        pl.kernel,
        mesh=mesh,
        out_type=jax.ShapeDtypeStruct((total, EMBED), jnp.float32),
        scratch_types=[
            pltpu.VMEM((irows_per_w, IDXW), jnp.int32),
            pltpu.VMEM((NBUF, SEQ, EMBED), jnp.float32),
            pltpu.VMEM_SHARED((SEQ, EMBED), jnp.float32),
            [pltpu.SemaphoreType.DMA] * NBUF,
            [pltpu.SemaphoreType.DMA] * NBUF,
        ],
    )
    def emb(idx_hbm, tok_hbm, pos_hbm, out_hbm, idx_v, rows_v, pos_v,
            gsems, osems):
        wid = lax.axis_index("s") * NC + lax.axis_index("c")

        @pl.when(lax.axis_index("s") == 0)
        def _():
            pltpu.sync_copy(pos_hbm, pos_v)

        plsc.subcore_barrier()
        pltpu.sync_copy(idx_hbm.at[pl.ds(wid * irows_per_w, irows_per_w)],
                        idx_v)

        def start_chunk(c, b):
            irow = c * (SEQ // IDXW)
            pltpu.sync_copy(pos_v, rows_v.at[b])
            pltpu.async_copy(tok_hbm.at[idx_v.at[irow]],
                             rows_v.at[b].at[pl.ds(0, IDXW)],
                             gsems[b], add=True)
            pltpu.async_copy(tok_hbm.at[idx_v.at[irow + 1]],
                             rows_v.at[b].at[pl.ds(IDXW, IDXW)],
                             gsems[b], add=True)

        def finish_chunk(c, b):
            pltpu.make_async_copy(tok_hbm.at[idx_v.at[0]],
                                  rows_v.at[b].at[pl.ds(0, IDXW)],
                                  gsems[b]).wait()
            pltpu.make_async_copy(tok_hbm.at[idx_v.at[0]],
                                  rows_v.at[b].at[pl.ds(IDXW, IDXW)],
                                  gsems[b]).wait()
            pltpu.async_copy(rows_v.at[b],
                             out_hbm.at[pl.ds(wid * rows_per_w + c * SEQ,
                                              SEQ)],
                             osems[b])

        def wait_out(b):
            pltpu.make_async_copy(
                rows_v.at[b],
                out_hbm.at[pl.ds(wid * rows_per_w, SEQ)],
                osems[b]).wait()

        def loop_body(g, carry):
            for b in range(NBUF):
                c = g * NBUF + b

                @pl.when(g > 0)
                def _():
                    wait_out(b)

                start_chunk(c, b)
            for b in range(NBUF):
                finish_chunk(g * NBUF + b, b)
            return carry

        lax.fori_loop(0, seq_per_w // NBUF, loop_body, 0)
        for b in range(NBUF):
            wait_out(b)

    return emb(idx2d, token_table, pos_table)


def kernel(x, token_table, pos_table):
    batch, seq = x.shape
    total = batch * seq
    idx2d = x.reshape(total // IDXW, IDXW).astype(jnp.int32)
    out = _tok_pos_embed(idx2d, token_table, pos_table, total)
    return out.reshape(batch, seq, EMBED)
